# Initial kernel scaffold; baseline (speedup 1.0000x reference)
import jax
import jax.numpy as jnp
from jax.experimental import pallas as pl


def kernel(x, edge_index, edge_attr, c1_linW, c1_linB, c1_lin2W, c1_lin2B, c1_m1W, c1_m1B, c1_m2W, c1_m2B, c2_linW, c2_linB, c2_lin2W, c2_lin2B, c2_m1W, c2_m1B, c2_m2W, c2_m2B):
    raise NotImplementedError("write your pallas kernel here")


# speedup vs baseline: 1.2680x; 1.2680x over previous
import functools
import jax
import jax.numpy as jnp
from jax import lax
from jax.experimental import pallas as pl
from jax.experimental.pallas import tpu as pltpu
from jax.experimental.pallas import tpu_sc as plsc

_N = 50000
_E = 800000
_HALF = _E // 2
_DP = 128
_BM = 512
_NP = 50176
_EP = 802816
_B = 128
_NC, _NS = 2, 16
_R = 14080
_TRASH = _R
_RANGES = ((0, _R, 0), (_R, _R, 1), (2 * _R, _R, 0), (3 * _R, _NP - 3 * _R, 1))


def _mm(a, wt, b, act):
    m = a.shape[0]

    def body(a_ref, w_ref, b_ref, o_ref):
        acc = jnp.dot(a_ref[...], w_ref[...],
                      preferred_element_type=jnp.float32) + b_ref[...]
        o_ref[...] = jnp.maximum(acc, 0.0) if act else acc

    return pl.pallas_call(
        body,
        grid=(m // _BM,),
        in_specs=[
            pl.BlockSpec((_BM, _DP), lambda i: (i, 0)),
            pl.BlockSpec((_DP, _DP), lambda i: (0, 0)),
            pl.BlockSpec((1, _DP), lambda i: (0, 0)),
        ],
        out_specs=pl.BlockSpec((_BM, _DP), lambda i: (i, 0)),
        out_shape=jax.ShapeDtypeStruct((m, _DP), jnp.float32),
    )(a, wt, b)


def _sc_messages(td, ts, et, idxd, idxs):
    per_tile = _EP // (_NC * _NS)
    nblk = per_tile // _B
    mesh = plsc.VectorSubcoreMesh(core_axis_name="c", subcore_axis_name="s")

    @functools.partial(
        pl.kernel, mesh=mesh,
        out_type=jax.ShapeDtypeStruct((_EP, _DP), jnp.float32),
        scratch_types=[
            pltpu.VMEM((_B,), jnp.int32),
            pltpu.VMEM((_B,), jnp.int32),
            pltpu.VMEM((_B, _DP), jnp.float32),
            pltpu.VMEM((_B, _DP), jnp.float32),
            pltpu.VMEM((_B, _DP), jnp.float32),
            pltpu.SemaphoreType.DMA,
            pltpu.SemaphoreType.DMA,
        ],
    )
    def k(td_h, ts_h, et_h, idxd_h, idxs_h, out_h, ivd, ivs, rd, rs, ev,
          sem1, sem2):
        wid = lax.axis_index("s") * _NC + lax.axis_index("c")
        base = wid * per_tile

        def blk(i, carry):
            off = pl.multiple_of(base + i * _B, _B)
            pltpu.sync_copy(idxd_h.at[pl.ds(off, _B)], ivd)
            pltpu.sync_copy(idxs_h.at[pl.ds(off, _B)], ivs)
            cp1 = pltpu.async_copy(td_h.at[ivd], rd, sem1)
            cp2 = pltpu.async_copy(ts_h.at[ivs], rs, sem2)
            pltpu.sync_copy(et_h.at[pl.ds(off, _B)], ev)
            cp1.wait()
            cp2.wait()

            def row(r, c2):
                for cj in range(_DP // 16):
                    sl = pl.ds(cj * 16, 16)
                    v = rd[r, sl] + rs[r, sl] + ev[r, sl]
                    ev[r, sl] = jnp.maximum(v, 0.0)
                return c2

            lax.fori_loop(0, _B, row, 0)
            pltpu.sync_copy(ev, out_h.at[pl.ds(off, _B)])
            return carry

        lax.fori_loop(0, nblk, blk, 0)

    return k(td, ts, et, idxd, idxs)


def _sc_scatter(msg, dst, xp):
    per_sub = _EP // _NS
    nblk = per_sub // _B
    mesh = plsc.VectorSubcoreMesh(core_axis_name="c", subcore_axis_name="s")

    @functools.partial(
        pl.kernel, mesh=mesh,
        out_type=jax.ShapeDtypeStruct((_NP, _DP), jnp.float32),
        scratch_types=[
            pltpu.VMEM_SHARED((_R + 8, _DP), jnp.float32),
            pltpu.VMEM((_B,), jnp.int32),
            pltpu.VMEM((_B,), jnp.int32),
            pltpu.VMEM((_B, _DP), jnp.float32),
        ],
    )
    def k(msg_h, dst_h, xp_h, out_h, shared, dv, ilv, mv):
        c = lax.axis_index("c")
        s = lax.axis_index("s")

        for rb, rl, rc in _RANGES:
            chunk = rl // _NS

            @pl.when(c == rc)
            def _():
                pltpu.sync_copy(xp_h.at[pl.ds(rb + s * chunk, chunk)],
                                shared.at[pl.ds(s * chunk, chunk)])
                plsc.subcore_barrier()

                def blk(i, carry):
                    off = pl.multiple_of(s * per_sub + i * _B, _B)
                    pltpu.sync_copy(dst_h.at[pl.ds(off, _B)], dv)
                    pltpu.sync_copy(msg_h.at[pl.ds(off, _B)], mv)
                    for cj in range(_B // 16):
                        sl = pl.ds(cj * 16, 16)
                        d = dv[sl] - rb
                        ok = (d >= 0) & (d < rl)
                        ilv[sl] = jnp.where(ok, d, _TRASH)
                    pltpu.sync_copy(mv, shared.at[ilv], add=True)
                    return carry

                lax.fori_loop(0, nblk, blk, 0)
                plsc.subcore_barrier()
                pltpu.sync_copy(shared.at[pl.ds(s * chunk, chunk)],
                                out_h.at[pl.ds(rb + s * chunk, chunk)])
                plsc.subcore_barrier()

    return k(msg, dst, xp)


def _padw(w):
    wt = w.T
    return jnp.pad(wt, ((0, _DP - wt.shape[0]), (0, _DP - wt.shape[1])))


def _padb(b):
    return jnp.pad(b, (0, _DP - b.shape[0]))[None, :]


def _conv(xp, idxd, idxs, dstp, ea1, ea2, linW, linB, lin2W, lin2B,
          m1W, m1B, m2W, m2B, relu_out):
    d = linW.shape[0]
    zb = jnp.zeros((1, _DP), jnp.float32)
    w1a, w1b, w1c = linW[:, :d], linW[:, d:2 * d], linW[:, 2 * d:]
    w2a, w2b, w2c = lin2W[:, :d], lin2W[:, d:2 * d], lin2W[:, 2 * d:]
    td = jnp.concatenate([_mm(xp, _padw(w1a), zb, False)[:_N],
                          _mm(xp, _padw(w2c), zb, False)[:_N]], axis=0)
    ts = jnp.concatenate([_mm(xp, _padw(w1c), zb, False)[:_N],
                          _mm(xp, _padw(w2a), zb, False)[:_N]], axis=0)
    et1 = _mm(ea1, _padw(w1b), _padb(linB), False)[:_HALF]
    et2 = _mm(ea2, _padw(w2b), _padb(lin2B), False)[:_HALF]
    et = jnp.concatenate(
        [et1, et2, jnp.zeros((_EP - _E, _DP), jnp.float32)], axis=0)
    msg = _sc_messages(td, ts, et, idxd, idxs)
    agg = _sc_scatter(msg, dstp, xp)
    h = _mm(agg, _padw(m1W), _padb(m1B), True)
    return _mm(h, _padw(m2W), _padb(m2B), relu_out)


def kernel(x, edge_index, edge_attr, c1_linW, c1_linB, c1_lin2W, c1_lin2B,
           c1_m1W, c1_m1B, c1_m2W, c1_m2B, c2_linW, c2_linB, c2_lin2W,
           c2_lin2B, c2_m1W, c2_m1B, c2_m2W, c2_m2B):
    d = x.shape[1]
    xp = jnp.pad(x, ((0, _NP - _N), (0, _DP - d)))
    src = edge_index[0]
    dst = edge_index[1]
    hoff = jnp.where(jnp.arange(_E, dtype=jnp.int32) < _HALF, 0, _N)
    idxd = jnp.pad(dst + hoff, (0, _EP - _E))
    idxs = jnp.pad(src + hoff, (0, _EP - _E))
    dstp = jnp.pad(dst, (0, _EP - _E), constant_values=-1)
    ea_rows_pad = -(-_HALF // _BM) * _BM - _HALF
    ea1 = jnp.pad(edge_attr[:_HALF], ((0, ea_rows_pad), (0, _DP - d)))
    ea2 = jnp.pad(edge_attr[_HALF:], ((0, ea_rows_pad), (0, _DP - d)))
    f = _conv(xp, idxd, idxs, dstp, ea1, ea2, c1_linW, c1_linB, c1_lin2W,
              c1_lin2B, c1_m1W, c1_m1B, c1_m2W, c1_m2B, True)
    out = _conv(f, idxd, idxs, dstp, ea1, ea2, c2_linW, c2_linB, c2_lin2W,
                c2_lin2B, c2_m1W, c2_m1B, c2_m2W, c2_m2B, False)
    return out[:_N]
